# Initial kernel scaffold; baseline (speedup 1.0000x reference)
import jax
import jax.numpy as jnp
from jax.experimental import pallas as pl


def kernel(x, emb, W1, b1, W2, b2, W3, b3):
    raise NotImplementedError("write your pallas kernel here")


# speedup vs baseline: 4.6106x; 4.6106x over previous
import functools

import jax
import jax.numpy as jnp
from jax import lax
from jax.experimental import pallas as pl
from jax.experimental.pallas import tpu as pltpu
from jax.experimental.pallas import tpu_sc as plsc

B = 16384
V = 1000000
D = 32
CTX = 6
H1 = 100
H2 = 10
O = 45

BF = B * CTX
NC = 2
NS = 16
NW = NC * NS
B_PER_W = BF // NW
CW = 128
NCHUNK = B_PER_W // CW


def _gather_kernel(idx_hbm, table_hbm, out_hbm, idx_v, rows_v, sem):
    wid = lax.axis_index("s") * NC + lax.axis_index("c")
    base = wid * B_PER_W
    pltpu.sync_copy(idx_hbm.at[wid], idx_v)

    def fire(j, carry):
        pltpu.async_copy(
            table_hbm.at[idx_v.at[j]],
            rows_v.at[pl.ds(j * CW, CW), :],
            sem,
        )
        return carry

    lax.fori_loop(0, NCHUNK, fire, 0)

    def drain(j, carry):
        pltpu.make_async_copy(
            table_hbm.at[idx_v.at[j]],
            rows_v.at[pl.ds(j * CW, CW), :],
            sem,
        ).wait()
        return carry

    lax.fori_loop(0, NCHUNK, drain, 0)

    pltpu.sync_copy(rows_v, out_hbm.at[pl.ds(base, B_PER_W), :])


@functools.cache
def _build_sc_gather():
    return functools.partial(
        pl.kernel,
        mesh=plsc.VectorSubcoreMesh(
            core_axis_name="c", subcore_axis_name="s",
            num_cores=NC, num_subcores=NS),
        out_type=jax.ShapeDtypeStruct((BF, D), jnp.float32),
        scratch_types=[
            pltpu.VMEM((NCHUNK, CW), jnp.int32),
            pltpu.VMEM((B_PER_W, D), jnp.float32),
            pltpu.SemaphoreType.DMA,
        ],
        compiler_params=pltpu.CompilerParams(use_tc_tiling_on_sc=False),
    )(_gather_kernel)


def _mlp_kernel(h_ref, w1_ref, b1_ref, w2_ref, b2_ref, w3_ref, b3_ref, o_ref):
    h = h_ref[...]
    h1 = lax.dot_general(h, w1_ref[...], (((1,), (0,)), ((), ())),
                         preferred_element_type=jnp.float32)
    h1 = jnp.maximum(h1 + b1_ref[...], 0.0)
    h2 = lax.dot_general(h1, w2_ref[...], (((1,), (0,)), ((), ())),
                         preferred_element_type=jnp.float32)
    h2 = jnp.maximum(h2 + b2_ref[...], 0.0)
    logits = lax.dot_general(h2, w3_ref[...], (((1,), (0,)), ((), ())),
                             preferred_element_type=jnp.float32)
    logits = logits + b3_ref[...]
    m = jnp.max(logits, axis=1, keepdims=True)
    z = logits - m
    lse = jnp.log(jnp.sum(jnp.exp(z), axis=1, keepdims=True))
    o_ref[...] = z - lse


_MLP_BLOCK = 2048


def _mlp(h, W1, b1, W2, b2, W3, b3):
    grid = (B // _MLP_BLOCK,)
    full = lambda i: (0, 0)
    return pl.pallas_call(
        _mlp_kernel,
        grid=grid,
        in_specs=[
            pl.BlockSpec((_MLP_BLOCK, D * CTX), lambda i: (i, 0)),
            pl.BlockSpec((D * CTX, H1), full),
            pl.BlockSpec((1, H1), full),
            pl.BlockSpec((H1, H2), full),
            pl.BlockSpec((1, H2), full),
            pl.BlockSpec((H2, O), full),
            pl.BlockSpec((1, O), full),
        ],
        out_specs=pl.BlockSpec((_MLP_BLOCK, O), lambda i: (i, 0)),
        out_shape=jax.ShapeDtypeStruct((B, O), jnp.float32),
    )(h, W1, b1, W2, b2, W3, b3)


def kernel(x, emb, W1, b1, W2, b2, W3, b3):
    idx = x.reshape(NW, NCHUNK, CW).astype(jnp.int32)
    gathered = _build_sc_gather()(idx, emb)
    h = gathered.reshape(B, CTX * D)
    return _mlp(h, W1, b1.reshape(1, H1), W2, b2.reshape(1, H2),
                W3, b3.reshape(1, O))
